# initial kernel scaffold (unmeasured)
import jax
import jax.numpy as jnp
from jax import lax
from jax.experimental import pallas as pl
from jax.experimental.pallas import tpu as pltpu

N_DEV = 4


def kernel(x, dest):
    m_per, n = x.shape
    dest2 = dest.reshape(32, 128)
    dm, dn = dest2.shape

    def body(
        x_ref,
        d_ref,
        xg_ref,
        dg_ref,
        copy_sems,
        send_sems,
        recv_sems,
        dsend_sems,
        drecv_sems,
    ):
        me = lax.axis_index("i")
        left = lax.rem(me + N_DEV - 1, N_DEV)
        right = lax.rem(me + 1, N_DEV)

        barrier_sem = pltpu.get_barrier_semaphore()
        for nbr in (left, right):
            pl.semaphore_signal(
                barrier_sem,
                inc=1,
                device_id=(nbr,),
                device_id_type=pl.DeviceIdType.MESH,
            )
        pl.semaphore_wait(barrier_sem, 2)

        cx = pltpu.make_async_copy(
            x_ref, xg_ref.at[pl.ds(me * m_per, m_per)], copy_sems.at[0]
        )
        cd = pltpu.make_async_copy(d_ref, dg_ref.at[me], copy_sems.at[1])
        cx.start()
        cd.start()
        cx.wait()
        cd.wait()

        for h in range(N_DEV - 1):
            org = lax.rem(me + N_DEV - h, N_DEV)
            rx = pltpu.make_async_remote_copy(
                src_ref=xg_ref.at[pl.ds(org * m_per, m_per)],
                dst_ref=xg_ref.at[pl.ds(org * m_per, m_per)],
                send_sem=send_sems.at[h],
                recv_sem=recv_sems.at[h],
                device_id=(right,),
                device_id_type=pl.DeviceIdType.MESH,
            )
            rd = pltpu.make_async_remote_copy(
                src_ref=dg_ref.at[org],
                dst_ref=dg_ref.at[org],
                send_sem=dsend_sems.at[h],
                recv_sem=drecv_sems.at[h],
                device_id=(right,),
                device_id_type=pl.DeviceIdType.MESH,
            )
            rx.start()
            rd.start()
            rx.wait()
            rd.wait()

    xg, dg = pl.pallas_call(
        body,
        out_shape=[
            jax.ShapeDtypeStruct((N_DEV * m_per, n), x.dtype),
            jax.ShapeDtypeStruct((N_DEV, dm, dn), dest2.dtype),
        ],
        in_specs=[
            pl.BlockSpec(memory_space=pltpu.ANY),
            pl.BlockSpec(memory_space=pltpu.ANY),
        ],
        out_specs=[
            pl.BlockSpec(memory_space=pltpu.ANY),
            pl.BlockSpec(memory_space=pltpu.ANY),
        ],
        scratch_shapes=[
            pltpu.SemaphoreType.DMA((2,)),
            pltpu.SemaphoreType.DMA((N_DEV - 1,)),
            pltpu.SemaphoreType.DMA((N_DEV - 1,)),
            pltpu.SemaphoreType.DMA((N_DEV - 1,)),
            pltpu.SemaphoreType.DMA((N_DEV - 1,)),
        ],
        compiler_params=pltpu.CompilerParams(collective_id=0),
    )(x, dest2)

    me = lax.axis_index("i")
    order = jnp.argsort(dg.reshape(-1), stable=True)
    idx = lax.dynamic_slice(order, (me * m_per,), (m_per,))
    return jnp.take(xg, idx, axis=0)


# baseline (device time: 838449 ns/iter reference)
import jax
import jax.numpy as jnp
from jax import lax
from jax.experimental import pallas as pl
from jax.experimental.pallas import tpu as pltpu

N_DEV = 4


def kernel(x, dest):
    m_per, n = x.shape
    dest2 = dest.reshape(32, 128)
    dm, dn = dest2.shape

    def body(
        x_ref,
        d_ref,
        xg_ref,
        dg_ref,
        copy_sems,
        send_sems,
        recv_sems,
        dsend_sems,
        drecv_sems,
    ):
        me = lax.axis_index("i")
        left = lax.rem(me + N_DEV - 1, N_DEV)
        right = lax.rem(me + 1, N_DEV)

        barrier_sem = pltpu.get_barrier_semaphore()
        for nbr in (left, right):
            pl.semaphore_signal(
                barrier_sem,
                inc=1,
                device_id=(nbr,),
                device_id_type=pl.DeviceIdType.MESH,
            )
        pl.semaphore_wait(barrier_sem, 2)

        cx = pltpu.make_async_copy(
            x_ref, xg_ref.at[pl.ds(me * m_per, m_per)], copy_sems.at[0]
        )
        cd = pltpu.make_async_copy(d_ref, dg_ref.at[me], copy_sems.at[1])
        cx.start()
        cd.start()
        cx.wait()
        cd.wait()

        for h in range(N_DEV - 1):
            org = lax.rem(me + N_DEV - h, N_DEV)
            rx = pltpu.make_async_remote_copy(
                src_ref=xg_ref.at[pl.ds(org * m_per, m_per)],
                dst_ref=xg_ref.at[pl.ds(org * m_per, m_per)],
                send_sem=send_sems.at[h],
                recv_sem=recv_sems.at[h],
                device_id=(right,),
                device_id_type=pl.DeviceIdType.MESH,
            )
            rd = pltpu.make_async_remote_copy(
                src_ref=dg_ref.at[org],
                dst_ref=dg_ref.at[org],
                send_sem=dsend_sems.at[h],
                recv_sem=drecv_sems.at[h],
                device_id=(right,),
                device_id_type=pl.DeviceIdType.MESH,
            )
            rx.start()
            rd.start()
            rx.wait()
            rd.wait()

    xg, dg = pl.pallas_call(
        body,
        out_shape=[
            jax.ShapeDtypeStruct((N_DEV * m_per, n), x.dtype),
            jax.ShapeDtypeStruct((N_DEV, dm, dn), dest2.dtype),
        ],
        in_specs=[
            pl.BlockSpec(memory_space=pl.ANY),
            pl.BlockSpec(memory_space=pl.ANY),
        ],
        out_specs=[
            pl.BlockSpec(memory_space=pl.ANY),
            pl.BlockSpec(memory_space=pl.ANY),
        ],
        scratch_shapes=[
            pltpu.SemaphoreType.DMA((2,)),
            pltpu.SemaphoreType.DMA((N_DEV - 1,)),
            pltpu.SemaphoreType.DMA((N_DEV - 1,)),
            pltpu.SemaphoreType.DMA((N_DEV - 1,)),
            pltpu.SemaphoreType.DMA((N_DEV - 1,)),
        ],
        compiler_params=pltpu.CompilerParams(collective_id=0),
    )(x, dest2)

    me = lax.axis_index("i")
    order = jnp.argsort(dg.reshape(-1), stable=True)
    idx = lax.dynamic_slice(order, (me * m_per,), (m_per,))
    return jnp.take(xg, idx, axis=0)


# device time: 171410 ns/iter; 4.8915x vs baseline; 4.8915x over previous
import jax
import jax.numpy as jnp
from jax import lax
from jax.experimental import pallas as pl
from jax.experimental.pallas import tpu as pltpu

N_DEV = 4


def _dest_allgather(dest2):
    dm, dn = dest2.shape

    def body(d_ref, dg_ref, copy_sem, send_sems, recv_sems):
        me = lax.axis_index("i")
        left = lax.rem(me + N_DEV - 1, N_DEV)
        right = lax.rem(me + 1, N_DEV)

        barrier_sem = pltpu.get_barrier_semaphore()
        for nbr in (left, right):
            pl.semaphore_signal(
                barrier_sem,
                inc=1,
                device_id=(nbr,),
                device_id_type=pl.DeviceIdType.MESH,
            )
        pl.semaphore_wait(barrier_sem, 2)

        cd = pltpu.make_async_copy(d_ref, dg_ref.at[me], copy_sem)
        cd.start()
        cd.wait()

        for h in range(N_DEV - 1):
            org = lax.rem(me + N_DEV - h, N_DEV)
            rd = pltpu.make_async_remote_copy(
                src_ref=dg_ref.at[org],
                dst_ref=dg_ref.at[org],
                send_sem=send_sems.at[h],
                recv_sem=recv_sems.at[h],
                device_id=(right,),
                device_id_type=pl.DeviceIdType.MESH,
            )
            rd.start()
            rd.wait()

    return pl.pallas_call(
        body,
        out_shape=jax.ShapeDtypeStruct((N_DEV, dm, dn), dest2.dtype),
        in_specs=[pl.BlockSpec(memory_space=pl.ANY)],
        out_specs=pl.BlockSpec(memory_space=pl.ANY),
        scratch_shapes=[
            pltpu.SemaphoreType.DMA,
            pltpu.SemaphoreType.DMA((N_DEV - 1,)),
            pltpu.SemaphoreType.DMA((N_DEV - 1,)),
        ],
        compiler_params=pltpu.CompilerParams(collective_id=0),
    )(dest2)


def kernel(x, dest):
    m_per, n = x.shape
    dest2 = dest.reshape(32, 128)

    dest_full = _dest_allgather(dest2).reshape(-1)

    me = lax.axis_index("i")
    ranks = jnp.arange(N_DEV, dtype=jnp.int32)
    oh_full = (dest_full[:, None] == ranks[None, :]).astype(jnp.int32)
    g0 = me * m_per
    before = (jnp.arange(N_DEV * m_per, dtype=jnp.int32) < g0).astype(
        jnp.int32
    )[:, None]
    base = jnp.sum(oh_full * before, axis=0).astype(jnp.int32)
    cnt_sd = jnp.sum(oh_full.reshape(N_DEV, m_per, N_DEV), axis=1)
    cnt_in = lax.dynamic_slice(cnt_sd, (0, me), (N_DEV, 1)).reshape(
        N_DEV
    ).astype(jnp.int32)
    oh_loc = lax.dynamic_slice(oh_full, (g0, 0), (m_per, N_DEV))
    excl = jnp.cumsum(oh_loc, axis=0) - oh_loc
    off = jnp.sum(oh_loc * (excl + base[None, :]), axis=1).astype(jnp.int32)
    dloc = dest.astype(jnp.int32)

    def body(d_sm, off_sm, cnt_sm, x_ref, out_ref, send_sem, loc_sem, recv_sems):
        my = lax.axis_index("i")

        barrier_sem = pltpu.get_barrier_semaphore()
        for k in range(1, N_DEV):
            pl.semaphore_signal(
                barrier_sem,
                inc=1,
                device_id=(lax.rem(my + k, N_DEV),),
                device_id_type=pl.DeviceIdType.MESH,
            )
        pl.semaphore_wait(barrier_sem, N_DEV - 1)

        def issue(j, carry):
            d = d_sm[j]
            o = off_sm[j]

            @pl.when(d == my)
            def _():
                cp = pltpu.make_async_copy(
                    x_ref.at[pl.ds(j, 1)], out_ref.at[pl.ds(o, 1)], loc_sem
                )
                cp.start()

            @pl.when(d != my)
            def _():
                rdma = pltpu.make_async_remote_copy(
                    src_ref=x_ref.at[pl.ds(j, 1)],
                    dst_ref=out_ref.at[pl.ds(o, 1)],
                    send_sem=send_sem,
                    recv_sem=recv_sems.at[my],
                    device_id=(d,),
                    device_id_type=pl.DeviceIdType.MESH,
                )
                rdma.start()

            return carry

        lax.fori_loop(0, m_per, issue, 0)

        n_loc = cnt_sm[my]
        n_rem = m_per - n_loc

        def wait_loc(j, carry):
            pltpu.make_async_copy(
                x_ref.at[pl.ds(0, 1)], out_ref.at[pl.ds(0, 1)], loc_sem
            ).wait()
            return carry

        lax.fori_loop(0, n_loc, wait_loc, 0)

        def wait_send(j, carry):
            pltpu.make_async_remote_copy(
                src_ref=x_ref.at[pl.ds(0, 1)],
                dst_ref=out_ref.at[pl.ds(0, 1)],
                send_sem=send_sem,
                recv_sem=recv_sems.at[my],
                device_id=(my,),
                device_id_type=pl.DeviceIdType.MESH,
            ).wait_send()
            return carry

        lax.fori_loop(0, n_rem, wait_send, 0)

        for s in range(N_DEV):
            expect = jnp.where(s == my, 0, cnt_sm[s])

            def wait_recv(j, carry, s=s):
                pltpu.make_async_remote_copy(
                    src_ref=x_ref.at[pl.ds(0, 1)],
                    dst_ref=out_ref.at[pl.ds(0, 1)],
                    send_sem=send_sem,
                    recv_sem=recv_sems.at[s],
                    device_id=(my,),
                    device_id_type=pl.DeviceIdType.MESH,
                ).wait_recv()
                return carry

            lax.fori_loop(0, expect, wait_recv, 0)

    return pl.pallas_call(
        body,
        out_shape=jax.ShapeDtypeStruct((m_per, n), x.dtype),
        in_specs=[
            pl.BlockSpec(memory_space=pltpu.SMEM),
            pl.BlockSpec(memory_space=pltpu.SMEM),
            pl.BlockSpec(memory_space=pltpu.SMEM),
            pl.BlockSpec(memory_space=pl.ANY),
        ],
        out_specs=pl.BlockSpec(memory_space=pl.ANY),
        scratch_shapes=[
            pltpu.SemaphoreType.DMA,
            pltpu.SemaphoreType.DMA,
            pltpu.SemaphoreType.DMA((N_DEV,)),
        ],
        compiler_params=pltpu.CompilerParams(collective_id=1),
    )(dloc, off, cnt_in, x)
